# baseline (speedup 1.0000x reference)
import functools

import jax
import jax.numpy as jnp
from jax import lax
from jax.experimental import pallas as pl
from jax.experimental.pallas import tpu as pltpu
from jax.experimental.pallas import tpu_sc as plsc

N = 32768
M = 2048
C = 32
R1SQ = 0.08 * 0.08
R2SQ = 0.16 * 0.16
NS1 = 16
NS2 = 32
DPAD = 48
BNS = (1.0 + 1e-3) ** -0.5

NW = 32
QPW = M // NW
NCHUNK = N // 16
SUPV = 32
GCH = 128
LOG1 = 48
LOG2 = 64


def _sc_body(xh, yh, zh, qxh, qyh, qzh, table, g1, g2, cnt1, cnt2,
             xv, yv, zv, qx, qy, qz, buf1c, buf2c, log1, log2,
             c1v, c2v, st1, st2, sj1, sj2, dflag,
             rowbuf, rowbuf2, sem, sem2):
    wid = lax.axis_index("s") * 2 + lax.axis_index("c")
    base = wid * QPW

    pltpu.sync_copy(xh, xv)
    pltpu.sync_copy(yh, yv)
    pltpu.sync_copy(zh, zv)
    pltpu.sync_copy(qxh.at[pl.ds(base * 16, QPW * 16)], qx)
    pltpu.sync_copy(qyh.at[pl.ds(base * 16, QPW * 16)], qy)
    pltpu.sync_copy(qzh.at[pl.ds(base * 16, QPW * 16)], qz)

    z16 = jnp.zeros((16,), jnp.int32)

    def zinit1(i, carry):
        buf1c[i >> 3, pl.ds((i & 7) * 16, 16)] = z16
        return carry

    def zinit2(i, carry):
        buf2c[i >> 3, pl.ds((i & 7) * 16, 16)] = z16
        return carry

    lax.fori_loop(0, QPW * NS1 // 16, zinit1, 0)
    lax.fori_loop(0, QPW * NS2 // 16, zinit2, 0)

    lane = lax.broadcasted_iota(jnp.int32, (16,), 0)

    def per_query(q, carry):
        qxv = qx[pl.ds(q * 16, 16)]
        qyv = qy[pl.ds(q * 16, 16)]
        qzv = qz[pl.ds(q * 16, 16)]
        st1[...] = z16
        st2[...] = z16
        sj1[...] = z16
        sj2[...] = z16
        dflag[0] = 0

        def super_body(sc, carry2):
            @pl.when(dflag[0] == 0)
            def _():
                def vec_body(v, state):
                    a1, a2, j1, j2 = state
                    c = sc * SUPV + v
                    dx = xv[pl.ds(c * 16, 16)] - qxv
                    dy = yv[pl.ds(c * 16, 16)] - qyv
                    dz = zv[pl.ds(c * 16, 16)] - qzv
                    d2 = dx * dx + dy * dy + dz * dz
                    gidx = c * 16 + lane
                    m1 = d2 < R1SQ
                    m2 = d2 < R2SQ
                    p1 = plsc.all_reduce_population_count(m1)
                    p2 = plsc.all_reduce_population_count(m2)
                    e1 = jnp.where(m1, gidx, -1)
                    e2 = jnp.where(m2, gidx, -1)
                    plsc.store_scatter(
                        log1, [jnp.minimum(j1, LOG1 - 1) * 16 + lane], e1,
                        mask=j1 < LOG1)
                    plsc.store_scatter(
                        log2, [jnp.minimum(j2, LOG2 - 1) * 16 + lane], e2,
                        mask=j2 < LOG2)
                    return (a1 + p1, a2 + p2,
                            j1 + jnp.where(p1 > 0, 1, 0),
                            j2 + jnp.where(p2 > 0, 1, 0))

                r1, r2, rj1, rj2 = lax.fori_loop(
                    0, SUPV, vec_body,
                    (st1[...], st2[...], sj1[...], sj2[...]))
                st1[...] = r1
                st2[...] = r2
                sj1[...] = rj1
                sj2[...] = rj2
                n1s = jnp.sum(r1) >> 4
                n2s = jnp.sum(r2) >> 4
                dflag[0] = jnp.where((n1s >= NS1) & (n2s >= NS2), 1, 0)

            return carry2

        lax.fori_loop(0, NCHUNK // SUPV, super_body, 0)

        def compact(log, rows, cap, buf):
            def row_body(r, a):
                v = log[pl.ds(r * 16, 16)]
                m = v >= 0
                cum = plsc.cumsum(jnp.where(m, 1, 0))
                pos = a + cum - 1
                w = m & (pos < cap)
                p = jnp.clip(q * cap + pos, 0, QPW * cap - 1)
                plsc.store_scatter(buf, [p >> 7, p & 127], v, mask=w)
                return a + plsc.all_reduce_population_count(m)

            lax.fori_loop(0, rows, row_body, z16)

        compact(log1, jnp.sum(sj1[...]) >> 4, NS1, buf1c)
        compact(log2, jnp.sum(sj2[...]) >> 4, NS2, buf2c)
        qsplat = jnp.full((16,), q, jnp.int32)
        lane0 = lane == 0
        plsc.store_scatter(c1v, [qsplat], st1[...], mask=lane0)
        plsc.store_scatter(c2v, [qsplat], st2[...], mask=lane0)
        return carry

    lax.fori_loop(0, QPW, per_query, 0)

    rbs = (rowbuf, rowbuf2)
    sems = (sem, sem2)
    work = ([(buf1c, k, g1, base * NS1 + k * GCH)
             for k in range(QPW * NS1 // GCH)]
            + [(buf2c, k, g2, base * NS2 + k * GCH)
               for k in range(QPW * NS2 // GCH)])
    descs = [None, None]
    prev = None
    for j, (src, k, dst, off) in enumerate(work):
        descs[j % 2] = pltpu.async_copy(table.at[src.at[k]],
                                        rbs[j % 2], sems[j % 2])
        if prev is not None:
            pj, pdst, poff = prev
            descs[pj % 2].wait()
            pltpu.sync_copy(rbs[pj % 2], pdst.at[pl.ds(poff, GCH)])
        prev = (j, dst, off)
    pj, pdst, poff = prev
    descs[pj % 2].wait()
    pltpu.sync_copy(rbs[pj % 2], pdst.at[pl.ds(poff, GCH)])

    pltpu.sync_copy(c1v, cnt1.at[pl.ds(base, QPW)])
    pltpu.sync_copy(c2v, cnt2.at[pl.ds(base, QPW)])


@functools.cache
def _make_sc_ballq():
    return functools.partial(
        pl.kernel,
        out_type=(
        jax.ShapeDtypeStruct((M * NS1, DPAD), jnp.float32),
        jax.ShapeDtypeStruct((M * NS2, DPAD), jnp.float32),
        jax.ShapeDtypeStruct((M,), jnp.int32),
        jax.ShapeDtypeStruct((M,), jnp.int32),
    ),
    mesh=plsc.VectorSubcoreMesh(core_axis_name="c", subcore_axis_name="s"),
    compiler_params=pltpu.CompilerParams(needs_layout_passes=False,
                                         use_tc_tiling_on_sc=False),
    scratch_types=[
        pltpu.VMEM((N,), jnp.float32),
        pltpu.VMEM((N,), jnp.float32),
        pltpu.VMEM((N,), jnp.float32),
        pltpu.VMEM((QPW * 16,), jnp.float32),
        pltpu.VMEM((QPW * 16,), jnp.float32),
        pltpu.VMEM((QPW * 16,), jnp.float32),
        pltpu.VMEM((QPW * NS1 // GCH, GCH), jnp.int32),
        pltpu.VMEM((QPW * NS2 // GCH, GCH), jnp.int32),
        pltpu.VMEM((LOG1 * 16,), jnp.int32),
        pltpu.VMEM((LOG2 * 16,), jnp.int32),
        pltpu.VMEM((QPW,), jnp.int32),
        pltpu.VMEM((QPW,), jnp.int32),
        pltpu.VMEM((16,), jnp.int32),
        pltpu.VMEM((16,), jnp.int32),
        pltpu.VMEM((16,), jnp.int32),
        pltpu.VMEM((16,), jnp.int32),
        pltpu.SMEM((1,), jnp.int32),
        pltpu.VMEM((GCH, DPAD), jnp.float32),
        pltpu.VMEM((GCH, DPAD), jnp.float32),
        pltpu.SemaphoreType.DMA,
        pltpu.SemaphoreType.DMA,
    ],
)(_sc_body)


MB = 256


def _tc_body(g1, g2, q, c1, c2, w1a, w2a, w1b, w2b, wf, bn, bnf, out):
    def scale(gref, cref, w1ref, w2ref, brow, ns):
        w1 = w1ref[...]
        a = jnp.dot(gref[...], w1, precision=lax.Precision.HIGHEST,
                    preferred_element_type=jnp.float32)
        qp = jnp.dot(q[...], w1[0:3, :], precision=lax.Precision.HIGHEST,
                     preferred_element_type=jnp.float32)
        a3 = a.reshape(MB, ns, 32) - qp[:, None, :]
        s1 = (bn[brow:brow + 1, :] * BNS).reshape(1, 1, 32)
        b1 = bn[brow + 1:brow + 2, :].reshape(1, 1, 32)
        h = jnp.maximum(a3 * s1 + b1, 0.0)
        h2 = jnp.dot(h.reshape(MB * ns, 32), w2ref[...],
                     precision=lax.Precision.HIGHEST,
                     preferred_element_type=jnp.float32)
        h2 = jnp.maximum(h2 * (bn[brow + 2:brow + 3, :] * BNS)
                         + bn[brow + 3:brow + 4, :], 0.0)
        slot = lax.broadcasted_iota(jnp.int32, (MB * ns, 1), 0) % ns
        valid = slot < jnp.maximum(cref[...], 1)
        h2 = jnp.where(valid, h2, -1e30)
        return jnp.max(h2.reshape(MB, ns, 32), axis=1)

    p1 = scale(g1, c1, w1a, w2a, 0, NS1)
    p2 = scale(g2, c2, w1b, w2b, 4, NS2)
    f = jnp.concatenate([p1, p2], axis=1)
    ff = jnp.dot(f, wf[...], precision=lax.Precision.HIGHEST,
                 preferred_element_type=jnp.float32)
    ff = jnp.maximum(ff * (bnf[0:1, :] * BNS) + bnf[1:2, :], 0.0)
    out[...] = ff


def _tc_mlp(g1, g2, q, c1, c2, w1a, w2a, w1b, w2b, wf, bn, bnf):
    grid = M // MB
    full = lambda shape: pl.BlockSpec(shape, lambda i: (0, 0))
    return pl.pallas_call(
        _tc_body,
        grid=(grid,),
        in_specs=[
            pl.BlockSpec((MB * NS1, DPAD), lambda i: (i, 0)),
            pl.BlockSpec((MB * NS2, DPAD), lambda i: (i, 0)),
            pl.BlockSpec((MB, 3), lambda i: (i, 0)),
            pl.BlockSpec((MB * NS1, 1), lambda i: (i, 0)),
            pl.BlockSpec((MB * NS2, 1), lambda i: (i, 0)),
            full((DPAD, 32)), full((32, 32)),
            full((DPAD, 32)), full((32, 32)),
            full((64, 128)), full((8, 32)), full((2, 128)),
        ],
        out_specs=pl.BlockSpec((MB, 128), lambda i: (i, 0)),
        out_shape=jax.ShapeDtypeStruct((M, 128), jnp.float32),
    )(g1, g2, q, c1, c2, w1a, w2a, w1b, w2b, wf, bn, bnf)


def kernel(xyz, xyz_batch_cnt, new_xyz, new_xyz_batch_cnt, features, params):
    f32 = jnp.float32
    xh, yh, zh = (xyz[:, i] for i in range(3))
    qr = jnp.broadcast_to(new_xyz.T.reshape(3, M, 1), (3, M, 16)).reshape(3, M * 16)
    table = jnp.concatenate(
        [xyz, features, jnp.zeros((N, DPAD - 3 - C), f32)], axis=1)

    g1, g2, cnt1, cnt2 = _make_sc_ballq()(xh, yh, zh, qr[0], qr[1], qr[2], table)

    pa, pb = params["scales"]
    pad_w = lambda w: jnp.pad(w, ((0, 0), (0, DPAD - 3 - C))).T
    w1a, w1b = pad_w(pa["w1"]), pad_w(pb["w1"])
    w2a, w2b = pa["w2"].T, pb["w2"].T
    wf = params["fusion_w"].T
    bn = jnp.stack([pa["g1"], pa["b1"], pa["g2"], pa["b2"],
                    pb["g1"], pb["b1"], pb["g2"], pb["b2"]], axis=0)
    bnf = jnp.stack([params["fusion_g"], params["fusion_b"]], axis=0)

    c1rep = jnp.repeat(cnt1, NS1).reshape(M * NS1, 1)
    c2rep = jnp.repeat(cnt2, NS2).reshape(M * NS2, 1)

    f = _tc_mlp(g1, g2, new_xyz, c1rep, c2rep, w1a, w2a, w1b, w2b, wf, bn, bnf)
    return (new_xyz, f)


# speedup vs baseline: 1.0078x; 1.0078x over previous
import functools

import jax
import jax.numpy as jnp
from jax import lax
from jax.experimental import pallas as pl
from jax.experimental.pallas import tpu as pltpu
from jax.experimental.pallas import tpu_sc as plsc

N = 32768
M = 2048
C = 32
R1SQ = 0.08 * 0.08
R2SQ = 0.16 * 0.16
NS1 = 16
NS2 = 32
DPAD = 48
BNS = (1.0 + 1e-3) ** -0.5

NW = 32
QPW = M // NW
NCHUNK = N // 16
SUPV = 16
GCH = 128
LOG1 = 16
LOG2 = 32


def _sc_body(xh, yh, zh, qxh, qyh, qzh, table, g1, g2, cnt1, cnt2,
             xv, yv, zv, qx, qy, qz, buf1c, buf2c, log1, log2,
             c1v, c2v, st1, st2, sj1, sj2, dflag,
             rowbuf, rowbuf2, sem, sem2):
    wid = lax.axis_index("s") * 2 + lax.axis_index("c")
    base = wid * QPW

    pltpu.sync_copy(xh, xv)
    pltpu.sync_copy(yh, yv)
    pltpu.sync_copy(zh, zv)
    pltpu.sync_copy(qxh.at[pl.ds(base * 16, QPW * 16)], qx)
    pltpu.sync_copy(qyh.at[pl.ds(base * 16, QPW * 16)], qy)
    pltpu.sync_copy(qzh.at[pl.ds(base * 16, QPW * 16)], qz)

    z16 = jnp.zeros((16,), jnp.int32)

    def zinit1(i, carry):
        buf1c[i >> 3, pl.ds((i & 7) * 16, 16)] = z16
        return carry

    def zinit2(i, carry):
        buf2c[i >> 3, pl.ds((i & 7) * 16, 16)] = z16
        return carry

    lax.fori_loop(0, QPW * NS1 // 16, zinit1, 0)
    lax.fori_loop(0, QPW * NS2 // 16, zinit2, 0)

    lane = lax.broadcasted_iota(jnp.int32, (16,), 0)

    def per_query(q, carry):
        qxv = qx[pl.ds(q * 16, 16)]
        qyv = qy[pl.ds(q * 16, 16)]
        qzv = qz[pl.ds(q * 16, 16)]
        st1[...] = z16
        st2[...] = z16
        sj1[...] = z16
        sj2[...] = z16
        dflag[0] = 0

        def super_body(sc, carry2):
            @pl.when(dflag[0] == 0)
            def _():
                def vec_body(v, state):
                    a1, a2, j1, j2 = state
                    c = sc * SUPV + v
                    dx = xv[pl.ds(c * 16, 16)] - qxv
                    dy = yv[pl.ds(c * 16, 16)] - qyv
                    dz = zv[pl.ds(c * 16, 16)] - qzv
                    d2 = dx * dx + dy * dy + dz * dz
                    gidx = c * 16 + lane
                    m1 = d2 < R1SQ
                    m2 = d2 < R2SQ
                    p1 = plsc.all_reduce_population_count(m1)
                    p2 = plsc.all_reduce_population_count(m2)
                    e1 = jnp.where(m1, gidx, -1)
                    e2 = jnp.where(m2, gidx, -1)
                    plsc.store_scatter(
                        log1, [jnp.minimum(j1, LOG1 - 1) * 16 + lane], e1,
                        mask=j1 < LOG1)
                    plsc.store_scatter(
                        log2, [jnp.minimum(j2, LOG2 - 1) * 16 + lane], e2,
                        mask=j2 < LOG2)
                    return (a1 + p1, a2 + p2,
                            j1 + jnp.where(p1 > 0, 1, 0),
                            j2 + jnp.where(p2 > 0, 1, 0))

                r1, r2, rj1, rj2 = lax.fori_loop(
                    0, SUPV, vec_body,
                    (st1[...], st2[...], sj1[...], sj2[...]))
                st1[...] = r1
                st2[...] = r2
                sj1[...] = rj1
                sj2[...] = rj2
                n1s = jnp.sum(r1) >> 4
                n2s = jnp.sum(r2) >> 4
                dflag[0] = jnp.where((n1s >= NS1) & (n2s >= NS2), 1, 0)

            return carry2

        lax.fori_loop(0, NCHUNK // SUPV, super_body, 0)

        def compact(log, rows, cap, buf):
            def row_body(r, a):
                v = log[pl.ds(r * 16, 16)]
                m = v >= 0
                cum = plsc.cumsum(jnp.where(m, 1, 0))
                pos = a + cum - 1
                w = m & (pos < cap)
                p = jnp.clip(q * cap + pos, 0, QPW * cap - 1)
                plsc.store_scatter(buf, [p >> 7, p & 127], v, mask=w)
                return a + plsc.all_reduce_population_count(m)

            lax.fori_loop(0, rows, row_body, z16)

        compact(log1, jnp.minimum(jnp.sum(sj1[...]) >> 4, LOG1), NS1, buf1c)
        compact(log2, jnp.minimum(jnp.sum(sj2[...]) >> 4, LOG2), NS2, buf2c)
        qsplat = jnp.full((16,), q, jnp.int32)
        lane0 = lane == 0
        plsc.store_scatter(c1v, [qsplat], st1[...], mask=lane0)
        plsc.store_scatter(c2v, [qsplat], st2[...], mask=lane0)
        return carry

    lax.fori_loop(0, QPW, per_query, 0)

    rbs = (rowbuf, rowbuf2)
    sems = (sem, sem2)
    work = ([(buf1c, k, g1, base * NS1 + k * GCH)
             for k in range(QPW * NS1 // GCH)]
            + [(buf2c, k, g2, base * NS2 + k * GCH)
               for k in range(QPW * NS2 // GCH)])
    descs = [None, None]
    prev = None
    for j, (src, k, dst, off) in enumerate(work):
        descs[j % 2] = pltpu.async_copy(table.at[src.at[k]],
                                        rbs[j % 2], sems[j % 2])
        if prev is not None:
            pj, pdst, poff = prev
            descs[pj % 2].wait()
            pltpu.sync_copy(rbs[pj % 2], pdst.at[pl.ds(poff, GCH)])
        prev = (j, dst, off)
    pj, pdst, poff = prev
    descs[pj % 2].wait()
    pltpu.sync_copy(rbs[pj % 2], pdst.at[pl.ds(poff, GCH)])

    pltpu.sync_copy(c1v, cnt1.at[pl.ds(base, QPW)])
    pltpu.sync_copy(c2v, cnt2.at[pl.ds(base, QPW)])


@functools.cache
def _make_sc_ballq():
    return functools.partial(
        pl.kernel,
        out_type=(
        jax.ShapeDtypeStruct((M * NS1, DPAD), jnp.float32),
        jax.ShapeDtypeStruct((M * NS2, DPAD), jnp.float32),
        jax.ShapeDtypeStruct((M,), jnp.int32),
        jax.ShapeDtypeStruct((M,), jnp.int32),
    ),
    mesh=plsc.VectorSubcoreMesh(core_axis_name="c", subcore_axis_name="s"),
    compiler_params=pltpu.CompilerParams(needs_layout_passes=False,
                                         use_tc_tiling_on_sc=False),
    scratch_types=[
        pltpu.VMEM((N,), jnp.float32),
        pltpu.VMEM((N,), jnp.float32),
        pltpu.VMEM((N,), jnp.float32),
        pltpu.VMEM((QPW * 16,), jnp.float32),
        pltpu.VMEM((QPW * 16,), jnp.float32),
        pltpu.VMEM((QPW * 16,), jnp.float32),
        pltpu.VMEM((QPW * NS1 // GCH, GCH), jnp.int32),
        pltpu.VMEM((QPW * NS2 // GCH, GCH), jnp.int32),
        pltpu.VMEM((LOG1 * 16,), jnp.int32),
        pltpu.VMEM((LOG2 * 16,), jnp.int32),
        pltpu.VMEM((QPW,), jnp.int32),
        pltpu.VMEM((QPW,), jnp.int32),
        pltpu.VMEM((16,), jnp.int32),
        pltpu.VMEM((16,), jnp.int32),
        pltpu.VMEM((16,), jnp.int32),
        pltpu.VMEM((16,), jnp.int32),
        pltpu.SMEM((1,), jnp.int32),
        pltpu.VMEM((GCH, DPAD), jnp.float32),
        pltpu.VMEM((GCH, DPAD), jnp.float32),
        pltpu.SemaphoreType.DMA,
        pltpu.SemaphoreType.DMA,
    ],
)(_sc_body)


MB = 256


def _tc_body(g1, g2, q, c1, c2, w1a, w2a, w1b, w2b, wf, bn, bnf, out):
    def scale(gref, cref, w1ref, w2ref, brow, ns):
        w1 = w1ref[...]
        a = jnp.dot(gref[...], w1, precision=lax.Precision.HIGHEST,
                    preferred_element_type=jnp.float32)
        qp = jnp.dot(q[...], w1[0:3, :], precision=lax.Precision.HIGHEST,
                     preferred_element_type=jnp.float32)
        a3 = a.reshape(MB, ns, 32) - qp[:, None, :]
        s1 = (bn[brow:brow + 1, :] * BNS).reshape(1, 1, 32)
        b1 = bn[brow + 1:brow + 2, :].reshape(1, 1, 32)
        h = jnp.maximum(a3 * s1 + b1, 0.0)
        h2 = jnp.dot(h.reshape(MB * ns, 32), w2ref[...],
                     precision=lax.Precision.HIGHEST,
                     preferred_element_type=jnp.float32)
        h2 = jnp.maximum(h2 * (bn[brow + 2:brow + 3, :] * BNS)
                         + bn[brow + 3:brow + 4, :], 0.0)
        slot = lax.broadcasted_iota(jnp.int32, (MB * ns, 1), 0) % ns
        valid = slot < jnp.maximum(cref[...], 1)
        h2 = jnp.where(valid, h2, -1e30)
        return jnp.max(h2.reshape(MB, ns, 32), axis=1)

    p1 = scale(g1, c1, w1a, w2a, 0, NS1)
    p2 = scale(g2, c2, w1b, w2b, 4, NS2)
    f = jnp.concatenate([p1, p2], axis=1)
    ff = jnp.dot(f, wf[...], precision=lax.Precision.HIGHEST,
                 preferred_element_type=jnp.float32)
    ff = jnp.maximum(ff * (bnf[0:1, :] * BNS) + bnf[1:2, :], 0.0)
    out[...] = ff


def _tc_mlp(g1, g2, q, c1, c2, w1a, w2a, w1b, w2b, wf, bn, bnf):
    grid = M // MB
    full = lambda shape: pl.BlockSpec(shape, lambda i: (0, 0))
    return pl.pallas_call(
        _tc_body,
        grid=(grid,),
        in_specs=[
            pl.BlockSpec((MB * NS1, DPAD), lambda i: (i, 0)),
            pl.BlockSpec((MB * NS2, DPAD), lambda i: (i, 0)),
            pl.BlockSpec((MB, 3), lambda i: (i, 0)),
            pl.BlockSpec((MB * NS1, 1), lambda i: (i, 0)),
            pl.BlockSpec((MB * NS2, 1), lambda i: (i, 0)),
            full((DPAD, 32)), full((32, 32)),
            full((DPAD, 32)), full((32, 32)),
            full((64, 128)), full((8, 32)), full((2, 128)),
        ],
        out_specs=pl.BlockSpec((MB, 128), lambda i: (i, 0)),
        out_shape=jax.ShapeDtypeStruct((M, 128), jnp.float32),
    )(g1, g2, q, c1, c2, w1a, w2a, w1b, w2b, wf, bn, bnf)


def kernel(xyz, xyz_batch_cnt, new_xyz, new_xyz_batch_cnt, features, params):
    f32 = jnp.float32
    xh, yh, zh = (xyz[:, i] for i in range(3))
    qr = jnp.broadcast_to(new_xyz.T.reshape(3, M, 1), (3, M, 16)).reshape(3, M * 16)
    table = jnp.concatenate(
        [xyz, features, jnp.zeros((N, DPAD - 3 - C), f32)], axis=1)

    g1, g2, cnt1, cnt2 = _make_sc_ballq()(xh, yh, zh, qr[0], qr[1], qr[2], table)

    pa, pb = params["scales"]
    pad_w = lambda w: jnp.pad(w, ((0, 0), (0, DPAD - 3 - C))).T
    w1a, w1b = pad_w(pa["w1"]), pad_w(pb["w1"])
    w2a, w2b = pa["w2"].T, pb["w2"].T
    wf = params["fusion_w"].T
    bn = jnp.stack([pa["g1"], pa["b1"], pa["g2"], pa["b2"],
                    pb["g1"], pb["b1"], pb["g2"], pb["b2"]], axis=0)
    bnf = jnp.stack([params["fusion_g"], params["fusion_b"]], axis=0)

    c1rep = jnp.repeat(cnt1, NS1).reshape(M * NS1, 1)
    c2rep = jnp.repeat(cnt2, NS2).reshape(M * NS2, 1)

    f = _tc_mlp(g1, g2, new_xyz, c1rep, c2rep, w1a, w2a, w1b, w2b, wf, bn, bnf)
    return (new_xyz, f)


# speedup vs baseline: 1.0290x; 1.0210x over previous
import functools

import jax
import jax.numpy as jnp
from jax import lax
from jax.experimental import pallas as pl
from jax.experimental.pallas import tpu as pltpu
from jax.experimental.pallas import tpu_sc as plsc

N = 32768
M = 2048
C = 32
R1SQ = 0.08 * 0.08
R2SQ = 0.16 * 0.16
NS1 = 16
NS2 = 32
DPAD = 48
BNS = (1.0 + 1e-3) ** -0.5

NW = 32
QPW = M // NW
NCHUNK = N // 16
SUPV = 16
GCH = 128
LOG1 = 16
LOG2 = 32


def _sc_body(xh, yh, zh, qxh, qyh, qzh, table, g1, g2, cnt1, cnt2,
             xv, yv, zv, qx, qy, qz, buf1c, buf2c, log1, log2,
             c1v, c2v, st1, st2, sj1, sj2, dflag,
             rowbuf, rowbuf2, sem, sem2):
    wid = lax.axis_index("s") * 2 + lax.axis_index("c")
    base = wid * QPW

    pltpu.sync_copy(xh, xv)
    pltpu.sync_copy(yh, yv)
    pltpu.sync_copy(zh, zv)
    pltpu.sync_copy(qxh.at[pl.ds(base * 16, QPW * 16)], qx)
    pltpu.sync_copy(qyh.at[pl.ds(base * 16, QPW * 16)], qy)
    pltpu.sync_copy(qzh.at[pl.ds(base * 16, QPW * 16)], qz)

    z16 = jnp.zeros((16,), jnp.int32)

    def zinit1(i, carry):
        buf1c[i >> 3, pl.ds((i & 7) * 16, 16)] = z16
        return carry

    def zinit2(i, carry):
        buf2c[i >> 3, pl.ds((i & 7) * 16, 16)] = z16
        return carry

    lax.fori_loop(0, QPW * NS1 // 16, zinit1, 0)
    lax.fori_loop(0, QPW * NS2 // 16, zinit2, 0)

    lane = lax.broadcasted_iota(jnp.int32, (16,), 0)

    def per_query(q, carry):
        qxv = qx[pl.ds(q * 16, 16)]
        qyv = qy[pl.ds(q * 16, 16)]
        qzv = qz[pl.ds(q * 16, 16)]
        st1[...] = z16
        st2[...] = z16
        sj1[...] = z16
        sj2[...] = z16
        dflag[0] = 0

        def super_body(sc, carry2):
            @pl.when(dflag[0] == 0)
            def _():
                def vec_body(v, state):
                    a1, a2, j1, j2, gidx = state
                    c = sc * SUPV + v
                    dx = xv[pl.ds(c * 16, 16)] - qxv
                    dy = yv[pl.ds(c * 16, 16)] - qyv
                    dz = zv[pl.ds(c * 16, 16)] - qzv
                    d2 = dx * dx + dy * dy + dz * dz
                    m1 = d2 < R1SQ
                    m2 = d2 < R2SQ
                    p1 = plsc.all_reduce_population_count(m1)
                    p2 = plsc.all_reduce_population_count(m2)
                    e1 = jnp.where(m1, gidx, -1)
                    e2 = jnp.where(m2, gidx, -1)
                    plsc.store_scatter(
                        log1, [jnp.minimum(j1, LOG1 - 1) * 16 + lane], e1,
                        mask=j1 < LOG1)
                    plsc.store_scatter(
                        log2, [jnp.minimum(j2, LOG2 - 1) * 16 + lane], e2,
                        mask=j2 < LOG2)
                    return (a1 + p1, a2 + p2,
                            j1 + jnp.minimum(p1, 1),
                            j2 + jnp.minimum(p2, 1),
                            gidx + 16)

                r1, r2, rj1, rj2, _ = lax.fori_loop(
                    0, SUPV, vec_body,
                    (st1[...], st2[...], sj1[...], sj2[...],
                     sc * (SUPV * 16) + lane))
                st1[...] = r1
                st2[...] = r2
                sj1[...] = rj1
                sj2[...] = rj2
                n1s = jnp.sum(r1) >> 4
                n2s = jnp.sum(r2) >> 4
                dflag[0] = jnp.where((n1s >= NS1) & (n2s >= NS2), 1, 0)

            return carry2

        lax.fori_loop(0, NCHUNK // SUPV, super_body, 0)

        def compact(log, rows, cap, buf):
            def row_body(r, a):
                v = log[pl.ds(r * 16, 16)]
                m = v >= 0
                cum = plsc.cumsum(jnp.where(m, 1, 0))
                pos = a + cum - 1
                w = m & (pos < cap)
                p = jnp.clip(q * cap + pos, 0, QPW * cap - 1)
                plsc.store_scatter(buf, [p >> 7, p & 127], v, mask=w)
                return a + plsc.all_reduce_population_count(m)

            lax.fori_loop(0, rows, row_body, z16)

        compact(log1, jnp.minimum(jnp.sum(sj1[...]) >> 4, LOG1), NS1, buf1c)
        compact(log2, jnp.minimum(jnp.sum(sj2[...]) >> 4, LOG2), NS2, buf2c)
        qsplat = jnp.full((16,), q, jnp.int32)
        lane0 = lane == 0
        plsc.store_scatter(c1v, [qsplat], st1[...], mask=lane0)
        plsc.store_scatter(c2v, [qsplat], st2[...], mask=lane0)
        return carry

    lax.fori_loop(0, QPW, per_query, 0)

    rbs = (rowbuf, rowbuf2)
    sems = (sem, sem2)
    work = ([(buf1c, k, g1, base * NS1 + k * GCH)
             for k in range(QPW * NS1 // GCH)]
            + [(buf2c, k, g2, base * NS2 + k * GCH)
               for k in range(QPW * NS2 // GCH)])
    descs = [None, None]
    prev = None
    for j, (src, k, dst, off) in enumerate(work):
        descs[j % 2] = pltpu.async_copy(table.at[src.at[k]],
                                        rbs[j % 2], sems[j % 2])
        if prev is not None:
            pj, pdst, poff = prev
            descs[pj % 2].wait()
            pltpu.sync_copy(rbs[pj % 2], pdst.at[pl.ds(poff, GCH)])
        prev = (j, dst, off)
    pj, pdst, poff = prev
    descs[pj % 2].wait()
    pltpu.sync_copy(rbs[pj % 2], pdst.at[pl.ds(poff, GCH)])

    pltpu.sync_copy(c1v, cnt1.at[pl.ds(base, QPW)])
    pltpu.sync_copy(c2v, cnt2.at[pl.ds(base, QPW)])


@functools.cache
def _make_sc_ballq():
    return functools.partial(
        pl.kernel,
        out_type=(
        jax.ShapeDtypeStruct((M * NS1, DPAD), jnp.float32),
        jax.ShapeDtypeStruct((M * NS2, DPAD), jnp.float32),
        jax.ShapeDtypeStruct((M,), jnp.int32),
        jax.ShapeDtypeStruct((M,), jnp.int32),
    ),
    mesh=plsc.VectorSubcoreMesh(core_axis_name="c", subcore_axis_name="s"),
    compiler_params=pltpu.CompilerParams(needs_layout_passes=False,
                                         use_tc_tiling_on_sc=False),
    scratch_types=[
        pltpu.VMEM((N,), jnp.float32),
        pltpu.VMEM((N,), jnp.float32),
        pltpu.VMEM((N,), jnp.float32),
        pltpu.VMEM((QPW * 16,), jnp.float32),
        pltpu.VMEM((QPW * 16,), jnp.float32),
        pltpu.VMEM((QPW * 16,), jnp.float32),
        pltpu.VMEM((QPW * NS1 // GCH, GCH), jnp.int32),
        pltpu.VMEM((QPW * NS2 // GCH, GCH), jnp.int32),
        pltpu.VMEM((LOG1 * 16,), jnp.int32),
        pltpu.VMEM((LOG2 * 16,), jnp.int32),
        pltpu.VMEM((QPW,), jnp.int32),
        pltpu.VMEM((QPW,), jnp.int32),
        pltpu.VMEM((16,), jnp.int32),
        pltpu.VMEM((16,), jnp.int32),
        pltpu.VMEM((16,), jnp.int32),
        pltpu.VMEM((16,), jnp.int32),
        pltpu.SMEM((1,), jnp.int32),
        pltpu.VMEM((GCH, DPAD), jnp.float32),
        pltpu.VMEM((GCH, DPAD), jnp.float32),
        pltpu.SemaphoreType.DMA,
        pltpu.SemaphoreType.DMA,
    ],
)(_sc_body)


MB = 256


def _tc_body(g1, g2, q, c1, c2, w1a, w2a, w1b, w2b, wf, bn, bnf, out):
    def scale(gref, cref, w1ref, w2ref, brow, ns):
        w1 = w1ref[...]
        a = jnp.dot(gref[...], w1, precision=lax.Precision.HIGHEST,
                    preferred_element_type=jnp.float32)
        qp = jnp.dot(q[...], w1[0:3, :], precision=lax.Precision.HIGHEST,
                     preferred_element_type=jnp.float32)
        a3 = a.reshape(MB, ns, 32) - qp[:, None, :]
        s1 = (bn[brow:brow + 1, :] * BNS).reshape(1, 1, 32)
        b1 = bn[brow + 1:brow + 2, :].reshape(1, 1, 32)
        h = jnp.maximum(a3 * s1 + b1, 0.0)
        h2 = jnp.dot(h.reshape(MB * ns, 32), w2ref[...],
                     precision=lax.Precision.HIGHEST,
                     preferred_element_type=jnp.float32)
        h2 = jnp.maximum(h2 * (bn[brow + 2:brow + 3, :] * BNS)
                         + bn[brow + 3:brow + 4, :], 0.0)
        slot = lax.broadcasted_iota(jnp.int32, (MB * ns, 1), 0) % ns
        valid = slot < jnp.maximum(cref[...], 1)
        h2 = jnp.where(valid, h2, -1e30)
        return jnp.max(h2.reshape(MB, ns, 32), axis=1)

    p1 = scale(g1, c1, w1a, w2a, 0, NS1)
    p2 = scale(g2, c2, w1b, w2b, 4, NS2)
    f = jnp.concatenate([p1, p2], axis=1)
    ff = jnp.dot(f, wf[...], precision=lax.Precision.HIGHEST,
                 preferred_element_type=jnp.float32)
    ff = jnp.maximum(ff * (bnf[0:1, :] * BNS) + bnf[1:2, :], 0.0)
    out[...] = ff


def _tc_mlp(g1, g2, q, c1, c2, w1a, w2a, w1b, w2b, wf, bn, bnf):
    grid = M // MB
    full = lambda shape: pl.BlockSpec(shape, lambda i: (0, 0))
    return pl.pallas_call(
        _tc_body,
        grid=(grid,),
        in_specs=[
            pl.BlockSpec((MB * NS1, DPAD), lambda i: (i, 0)),
            pl.BlockSpec((MB * NS2, DPAD), lambda i: (i, 0)),
            pl.BlockSpec((MB, 3), lambda i: (i, 0)),
            pl.BlockSpec((MB * NS1, 1), lambda i: (i, 0)),
            pl.BlockSpec((MB * NS2, 1), lambda i: (i, 0)),
            full((DPAD, 32)), full((32, 32)),
            full((DPAD, 32)), full((32, 32)),
            full((64, 128)), full((8, 32)), full((2, 128)),
        ],
        out_specs=pl.BlockSpec((MB, 128), lambda i: (i, 0)),
        out_shape=jax.ShapeDtypeStruct((M, 128), jnp.float32),
    )(g1, g2, q, c1, c2, w1a, w2a, w1b, w2b, wf, bn, bnf)


def kernel(xyz, xyz_batch_cnt, new_xyz, new_xyz_batch_cnt, features, params):
    f32 = jnp.float32
    xh, yh, zh = (xyz[:, i] for i in range(3))
    qr = jnp.broadcast_to(new_xyz.T.reshape(3, M, 1), (3, M, 16)).reshape(3, M * 16)
    table = jnp.concatenate(
        [xyz, features, jnp.zeros((N, DPAD - 3 - C), f32)], axis=1)

    g1, g2, cnt1, cnt2 = _make_sc_ballq()(xh, yh, zh, qr[0], qr[1], qr[2], table)

    pa, pb = params["scales"]
    pad_w = lambda w: jnp.pad(w, ((0, 0), (0, DPAD - 3 - C))).T
    w1a, w1b = pad_w(pa["w1"]), pad_w(pb["w1"])
    w2a, w2b = pa["w2"].T, pb["w2"].T
    wf = params["fusion_w"].T
    bn = jnp.stack([pa["g1"], pa["b1"], pa["g2"], pa["b2"],
                    pb["g1"], pb["b1"], pb["g2"], pb["b2"]], axis=0)
    bnf = jnp.stack([params["fusion_g"], params["fusion_b"]], axis=0)

    c1rep = jnp.repeat(cnt1, NS1).reshape(M * NS1, 1)
    c2rep = jnp.repeat(cnt2, NS2).reshape(M * NS2, 1)

    f = _tc_mlp(g1, g2, new_xyz, c1rep, c2rep, w1a, w2a, w1b, w2b, wf, bn, bnf)
    return (new_xyz, f)


# speedup vs baseline: 1.0291x; 1.0001x over previous
import functools

import jax
import jax.numpy as jnp
from jax import lax
from jax.experimental import pallas as pl
from jax.experimental.pallas import tpu as pltpu
from jax.experimental.pallas import tpu_sc as plsc

N = 32768
M = 2048
C = 32
R1SQ = 0.08 * 0.08
R2SQ = 0.16 * 0.16
NS1 = 16
NS2 = 32
DPAD = 48
BNS = (1.0 + 1e-3) ** -0.5

NW = 32
QPW = M // NW
NCHUNK = N // 16
SUPV = 16
GCH = 128
LOG1 = 16
LOG2 = 32


def _sc_body(xh, yh, zh, qxh, qyh, qzh, table, g1, g2, cnt1, cnt2,
             xv, yv, zv, qx, qy, qz, buf1c, buf2c, log1, log2,
             c1v, c2v, st1, st2, sj1, sj2, dflag,
             rowbuf, rowbuf2, sem, sem2):
    wid = lax.axis_index("s") * 2 + lax.axis_index("c")
    base = wid * QPW

    pltpu.sync_copy(xh, xv)
    pltpu.sync_copy(yh, yv)
    pltpu.sync_copy(zh, zv)
    pltpu.sync_copy(qxh.at[pl.ds(base * 16, QPW * 16)], qx)
    pltpu.sync_copy(qyh.at[pl.ds(base * 16, QPW * 16)], qy)
    pltpu.sync_copy(qzh.at[pl.ds(base * 16, QPW * 16)], qz)

    z16 = jnp.zeros((16,), jnp.int32)

    def zinit1(i, carry):
        buf1c[i >> 3, pl.ds((i & 7) * 16, 16)] = z16
        return carry

    def zinit2(i, carry):
        buf2c[i >> 3, pl.ds((i & 7) * 16, 16)] = z16
        return carry

    lax.fori_loop(0, QPW * NS1 // 16, zinit1, 0)
    lax.fori_loop(0, QPW * NS2 // 16, zinit2, 0)

    lane = lax.broadcasted_iota(jnp.int32, (16,), 0)

    def per_query(q, carry):
        qxv = qx[pl.ds(q * 16, 16)]
        qyv = qy[pl.ds(q * 16, 16)]
        qzv = qz[pl.ds(q * 16, 16)]
        st1[...] = z16
        st2[...] = z16
        sj1[...] = z16
        sj2[...] = z16
        dflag[0] = 0

        def super_body(sc, carry2):
            @pl.when(dflag[0] == 0)
            def _():
                def vec_body(v, state):
                    a1, a2, j1, j2, gidx = state
                    c = sc * SUPV + v
                    dx = xv[pl.ds(c * 16, 16)] - qxv
                    dy = yv[pl.ds(c * 16, 16)] - qyv
                    dz = zv[pl.ds(c * 16, 16)] - qzv
                    d2 = dx * dx + dy * dy + dz * dz
                    m1 = d2 < R1SQ
                    m2 = d2 < R2SQ
                    p1 = plsc.all_reduce_population_count(m1)
                    p2 = plsc.all_reduce_population_count(m2)
                    e1 = jnp.where(m1, gidx, -1)
                    e2 = jnp.where(m2, gidx, -1)
                    plsc.store_scatter(
                        log1, [jnp.minimum(j1, LOG1 - 1) * 16 + lane], e1,
                        mask=j1 < LOG1)
                    plsc.store_scatter(
                        log2, [jnp.minimum(j2, LOG2 - 1) * 16 + lane], e2,
                        mask=j2 < LOG2)
                    return (a1 + p1, a2 + p2,
                            j1 + jnp.minimum(p1, 1),
                            j2 + jnp.minimum(p2, 1),
                            gidx + 16)

                r1, r2, rj1, rj2, _ = lax.fori_loop(
                    0, SUPV, vec_body,
                    (st1[...], st2[...], sj1[...], sj2[...],
                     sc * (SUPV * 16) + lane))
                st1[...] = r1
                st2[...] = r2
                sj1[...] = rj1
                sj2[...] = rj2
                ok = jnp.minimum(r1 - NS1, r2 - NS2)
                dflag[0] = jnp.where(jnp.max(ok) >= 0, 1, 0)

            return carry2

        lax.fori_loop(0, NCHUNK // SUPV, super_body, 0)

        def compact(log, rows, cap, buf):
            def row_body(r, a):
                v = log[pl.ds(r * 16, 16)]
                m = v >= 0
                cum = plsc.cumsum(jnp.where(m, 1, 0))
                pos = a + cum - 1
                w = m & (pos < cap)
                p = jnp.clip(q * cap + pos, 0, QPW * cap - 1)
                plsc.store_scatter(buf, [p >> 7, p & 127], v, mask=w)
                return a + plsc.all_reduce_population_count(m)

            lax.fori_loop(0, rows, row_body, z16)

        compact(log1, jnp.minimum(jnp.sum(sj1[...]) >> 4, LOG1), NS1, buf1c)
        compact(log2, jnp.minimum(jnp.sum(sj2[...]) >> 4, LOG2), NS2, buf2c)
        qsplat = jnp.full((16,), q, jnp.int32)
        lane0 = lane == 0
        plsc.store_scatter(c1v, [qsplat], st1[...], mask=lane0)
        plsc.store_scatter(c2v, [qsplat], st2[...], mask=lane0)
        return carry

    lax.fori_loop(0, QPW, per_query, 0)

    rbs = (rowbuf, rowbuf2)
    sems = (sem, sem2)
    work = ([(buf1c, k, g1, base * NS1 + k * GCH)
             for k in range(QPW * NS1 // GCH)]
            + [(buf2c, k, g2, base * NS2 + k * GCH)
               for k in range(QPW * NS2 // GCH)])
    descs = [None, None]
    prev = None
    for j, (src, k, dst, off) in enumerate(work):
        descs[j % 2] = pltpu.async_copy(table.at[src.at[k]],
                                        rbs[j % 2], sems[j % 2])
        if prev is not None:
            pj, pdst, poff = prev
            descs[pj % 2].wait()
            pltpu.sync_copy(rbs[pj % 2], pdst.at[pl.ds(poff, GCH)])
        prev = (j, dst, off)
    pj, pdst, poff = prev
    descs[pj % 2].wait()
    pltpu.sync_copy(rbs[pj % 2], pdst.at[pl.ds(poff, GCH)])

    pltpu.sync_copy(c1v, cnt1.at[pl.ds(base, QPW)])
    pltpu.sync_copy(c2v, cnt2.at[pl.ds(base, QPW)])


@functools.cache
def _make_sc_ballq():
    return functools.partial(
        pl.kernel,
        out_type=(
        jax.ShapeDtypeStruct((M * NS1, DPAD), jnp.float32),
        jax.ShapeDtypeStruct((M * NS2, DPAD), jnp.float32),
        jax.ShapeDtypeStruct((M,), jnp.int32),
        jax.ShapeDtypeStruct((M,), jnp.int32),
    ),
    mesh=plsc.VectorSubcoreMesh(core_axis_name="c", subcore_axis_name="s"),
    compiler_params=pltpu.CompilerParams(needs_layout_passes=False,
                                         use_tc_tiling_on_sc=False),
    scratch_types=[
        pltpu.VMEM((N,), jnp.float32),
        pltpu.VMEM((N,), jnp.float32),
        pltpu.VMEM((N,), jnp.float32),
        pltpu.VMEM((QPW * 16,), jnp.float32),
        pltpu.VMEM((QPW * 16,), jnp.float32),
        pltpu.VMEM((QPW * 16,), jnp.float32),
        pltpu.VMEM((QPW * NS1 // GCH, GCH), jnp.int32),
        pltpu.VMEM((QPW * NS2 // GCH, GCH), jnp.int32),
        pltpu.VMEM((LOG1 * 16,), jnp.int32),
        pltpu.VMEM((LOG2 * 16,), jnp.int32),
        pltpu.VMEM((QPW,), jnp.int32),
        pltpu.VMEM((QPW,), jnp.int32),
        pltpu.VMEM((16,), jnp.int32),
        pltpu.VMEM((16,), jnp.int32),
        pltpu.VMEM((16,), jnp.int32),
        pltpu.VMEM((16,), jnp.int32),
        pltpu.SMEM((1,), jnp.int32),
        pltpu.VMEM((GCH, DPAD), jnp.float32),
        pltpu.VMEM((GCH, DPAD), jnp.float32),
        pltpu.SemaphoreType.DMA,
        pltpu.SemaphoreType.DMA,
    ],
)(_sc_body)


MB = 256


def _tc_body(g1, g2, q, c1, c2, w1a, w2a, w1b, w2b, wf, bn, bnf, out):
    def scale(gref, cref, w1ref, w2ref, brow, ns):
        w1 = w1ref[...]
        a = jnp.dot(gref[...], w1, precision=lax.Precision.HIGHEST,
                    preferred_element_type=jnp.float32)
        qp = jnp.dot(q[...], w1[0:3, :], precision=lax.Precision.HIGHEST,
                     preferred_element_type=jnp.float32)
        a3 = a.reshape(MB, ns, 32) - qp[:, None, :]
        s1 = (bn[brow:brow + 1, :] * BNS).reshape(1, 1, 32)
        b1 = bn[brow + 1:brow + 2, :].reshape(1, 1, 32)
        h = jnp.maximum(a3 * s1 + b1, 0.0)
        h2 = jnp.dot(h.reshape(MB * ns, 32), w2ref[...],
                     precision=lax.Precision.HIGHEST,
                     preferred_element_type=jnp.float32)
        h2 = jnp.maximum(h2 * (bn[brow + 2:brow + 3, :] * BNS)
                         + bn[brow + 3:brow + 4, :], 0.0)
        slot = lax.broadcasted_iota(jnp.int32, (MB * ns, 1), 0) % ns
        valid = slot < jnp.maximum(cref[...], 1)
        h2 = jnp.where(valid, h2, -1e30)
        return jnp.max(h2.reshape(MB, ns, 32), axis=1)

    p1 = scale(g1, c1, w1a, w2a, 0, NS1)
    p2 = scale(g2, c2, w1b, w2b, 4, NS2)
    f = jnp.concatenate([p1, p2], axis=1)
    ff = jnp.dot(f, wf[...], precision=lax.Precision.HIGHEST,
                 preferred_element_type=jnp.float32)
    ff = jnp.maximum(ff * (bnf[0:1, :] * BNS) + bnf[1:2, :], 0.0)
    out[...] = ff


def _tc_mlp(g1, g2, q, c1, c2, w1a, w2a, w1b, w2b, wf, bn, bnf):
    grid = M // MB
    full = lambda shape: pl.BlockSpec(shape, lambda i: (0, 0))
    return pl.pallas_call(
        _tc_body,
        grid=(grid,),
        in_specs=[
            pl.BlockSpec((MB * NS1, DPAD), lambda i: (i, 0)),
            pl.BlockSpec((MB * NS2, DPAD), lambda i: (i, 0)),
            pl.BlockSpec((MB, 3), lambda i: (i, 0)),
            pl.BlockSpec((MB * NS1, 1), lambda i: (i, 0)),
            pl.BlockSpec((MB * NS2, 1), lambda i: (i, 0)),
            full((DPAD, 32)), full((32, 32)),
            full((DPAD, 32)), full((32, 32)),
            full((64, 128)), full((8, 32)), full((2, 128)),
        ],
        out_specs=pl.BlockSpec((MB, 128), lambda i: (i, 0)),
        out_shape=jax.ShapeDtypeStruct((M, 128), jnp.float32),
    )(g1, g2, q, c1, c2, w1a, w2a, w1b, w2b, wf, bn, bnf)


def kernel(xyz, xyz_batch_cnt, new_xyz, new_xyz_batch_cnt, features, params):
    f32 = jnp.float32
    xh, yh, zh = (xyz[:, i] for i in range(3))
    qr = jnp.broadcast_to(new_xyz.T.reshape(3, M, 1), (3, M, 16)).reshape(3, M * 16)
    table = jnp.concatenate(
        [xyz, features, jnp.zeros((N, DPAD - 3 - C), f32)], axis=1)

    g1, g2, cnt1, cnt2 = _make_sc_ballq()(xh, yh, zh, qr[0], qr[1], qr[2], table)

    pa, pb = params["scales"]
    pad_w = lambda w: jnp.pad(w, ((0, 0), (0, DPAD - 3 - C))).T
    w1a, w1b = pad_w(pa["w1"]), pad_w(pb["w1"])
    w2a, w2b = pa["w2"].T, pb["w2"].T
    wf = params["fusion_w"].T
    bn = jnp.stack([pa["g1"], pa["b1"], pa["g2"], pa["b2"],
                    pb["g1"], pb["b1"], pb["g2"], pb["b2"]], axis=0)
    bnf = jnp.stack([params["fusion_g"], params["fusion_b"]], axis=0)

    c1rep = jnp.repeat(cnt1, NS1).reshape(M * NS1, 1)
    c2rep = jnp.repeat(cnt2, NS2).reshape(M * NS2, 1)

    f = _tc_mlp(g1, g2, new_xyz, c1rep, c2rep, w1a, w2a, w1b, w2b, wf, bn, bnf)
    return (new_xyz, f)
